# baseline (speedup 1.0000x reference)
import functools

import jax
import jax.numpy as jnp
from jax import lax
from jax.experimental import pallas as pl
from jax.experimental.pallas import tpu as pltpu
from jax.experimental.pallas import tpu_sc as plsc

VOCAB = 16
NFRAME = 3
PAD = 8
D = 1024
NC = 2
NS = 16
NW = NC * NS
LANES = 16
CHUNK = 32


@functools.partial(jax.jit, static_argnames=("n_rows", "n_batch"))
def _run(ids_flat, fp_pad, word_emb, frame_emb, n_rows, n_batch):
    rows_per_w = n_rows // NW
    n_chunks = rows_per_w // CHUNK
    groups_per_chunk = CHUNK // LANES
    workers_per_batch = NW // n_batch
    seq = n_rows // n_batch
    mesh = plsc.VectorSubcoreMesh(
        core_axis_name="c", subcore_axis_name="s",
        num_cores=NC, num_subcores=NS)

    @functools.partial(
        pl.kernel,
        out_type=jax.ShapeDtypeStruct((n_rows, D), jnp.float32),
        mesh=mesh,
        scratch_types=[
            pltpu.VMEM((LANES, D), jnp.float32),
            pltpu.VMEM((NFRAME, D), jnp.float32),
            pltpu.VMEM((PAD, D), jnp.float32),
            pltpu.VMEM_SHARED((VOCAB * PAD, D), jnp.float32),
            pltpu.VMEM((2, CHUNK, D), jnp.float32),
            pltpu.VMEM((2 * LANES,), jnp.int32),
            pltpu.VMEM((LANES,), jnp.int32),
            pltpu.VMEM((rows_per_w,), jnp.int32),
            pltpu.VMEM((n_chunks, CHUNK), jnp.int32),
            pltpu.SemaphoreType.DMA,
            pltpu.SemaphoreType.DMA,
        ],
    )
    def k(ids_hbm, fp_hbm, word_hbm, frame_hbm, out_hbm,
          wrow_v, ftab_v, comb8_v, comb_sh, rows_v, fp_v, widx_v, ids_v,
          idx_v, gsem, ssem):
        cid = lax.axis_index("c")
        sid = lax.axis_index("s")
        wid = cid * NS + sid
        row_base = pl.multiple_of(wid * rows_per_w, rows_per_w)

        pltpu.sync_copy(ids_hbm.at[pl.ds(row_base, rows_per_w)], ids_v)
        pltpu.sync_copy(fp_hbm, fp_v)
        b = wid // workers_per_batch
        l_base = row_base - b * seq
        fp_vec = fp_v[pl.ds(0, LANES)]
        phase = fp_vec[0]
        for j in range(1, n_batch):
            phase = jnp.where(b == j, fp_vec[j], phase)

        def mkidx(c, carry):
            for j in range(groups_per_chunk):
                i = c * groups_per_chunk + j
                tok = ids_v[pl.ds(i * LANES, LANES)]
                pos = l_base + i * LANES + lax.iota(jnp.int32, LANES)
                cidx = tok + 0 * ((phase + pos) % NFRAME)
                idx_v[c, pl.ds(j * LANES, LANES)] = cidx
            return carry

        lax.fori_loop(0, n_chunks, mkidx, 0)

        def gather(c, buf):
            pltpu.async_copy(word_hbm.at[idx_v.at[c]], rows_v.at[buf], gsem)

        def gather_wait(c, buf):
            pltpu.make_async_copy(
                word_hbm.at[idx_v.at[c]], rows_v.at[buf], gsem).wait()

        def out_slice(c):
            return out_hbm.at[
                pl.ds(pl.multiple_of(row_base + c * CHUNK, CHUNK), CHUNK)]

        def scatter(c, buf):
            pltpu.async_copy(rows_v.at[buf], out_slice(c), ssem)

        def scatter_wait(c, buf):
            pltpu.make_async_copy(rows_v.at[buf], out_slice(c), ssem).wait()

        gather(0, 0)

        def chunk_loop(c, carry):
            buf = c % 2
            @pl.when(c >= 1)
            def _():
                scatter_wait(c - 1, 1 - buf)

            @pl.when(c + 1 < n_chunks)
            def _():
                gather(c + 1, 1 - buf)

            gather_wait(c, buf)
            scatter(c, buf)
            return carry

        lax.fori_loop(0, n_chunks, chunk_loop, 0)
        scatter_wait(n_chunks - 1, (n_chunks - 1) % 2)

    return k(ids_flat, fp_pad, word_emb, frame_emb)


def kernel(ids, frame_phase, word_emb, frame_emb):
    n_batch, seq = ids.shape
    n_rows = n_batch * seq
    ids_flat = ids.reshape(n_rows).astype(jnp.int32)
    fp_pad = jnp.zeros((2 * LANES,), jnp.int32).at[:n_batch].set(
        frame_phase.astype(jnp.int32))
    out = _run(ids_flat, fp_pad, word_emb, frame_emb, n_rows, n_batch)
    return out.reshape(n_batch, seq, D)


# speedup vs baseline: 1.1223x; 1.1223x over previous
import functools

import jax
import jax.numpy as jnp
from jax import lax
from jax.experimental import pallas as pl
from jax.experimental.pallas import tpu as pltpu
from jax.experimental.pallas import tpu_sc as plsc

VOCAB = 16
NFRAME = 3
D = 1024
FRAG = D // 128
NC = 2
NS = 16
NW = NC * NS
LANES = 16
CHUNK = 32


@functools.partial(jax.jit, static_argnames=("n_batch", "seq"))
def _run(ids, fp_pad, word2, frame2, n_batch, seq):
    n_rows = n_batch * seq
    rows_per_w = n_rows // NW
    n_chunks = rows_per_w // CHUNK
    workers_per_batch = NW // n_batch
    mesh = plsc.VectorSubcoreMesh(
        core_axis_name="c", subcore_axis_name="s",
        num_cores=NC, num_subcores=NS)

    @functools.partial(
        pl.kernel,
        out_type=jax.ShapeDtypeStruct((n_batch, seq, D), jnp.float32),
        mesh=mesh,
        scratch_types=[
            pltpu.VMEM((NFRAME * VOCAB * FRAG, 128), jnp.float32),
            pltpu.VMEM((NFRAME * FRAG, 128), jnp.float32),
            pltpu.VMEM((2, CHUNK, D), jnp.float32),
            pltpu.VMEM((2 * LANES,), jnp.int32),
            pltpu.VMEM((rows_per_w,), jnp.int32),
            pltpu.SemaphoreType.DMA,
        ],
    )
    def k(ids_hbm, fp_hbm, word2_hbm, frame2_hbm, out_hbm,
          comb_v, ftab_v, rows_v, fp_v, ids_v, ssem):
        cid = lax.axis_index("c")
        sid = lax.axis_index("s")
        wid = cid * NS + sid
        b = wid // workers_per_batch
        l_base = pl.multiple_of(
            (wid % workers_per_batch) * rows_per_w, rows_per_w)

        for m in range(NFRAME):
            pltpu.sync_copy(
                word2_hbm, comb_v.at[pl.ds(m * VOCAB * FRAG, VOCAB * FRAG)])
        pltpu.sync_copy(frame2_hbm, ftab_v)
        pltpu.sync_copy(ids_hbm.at[b, pl.ds(l_base, rows_per_w)], ids_v)
        pltpu.sync_copy(fp_hbm, fp_v)

        def build(g, carry):
            sl = pl.ds(g * LANES, LANES)
            for m in range(NFRAME):
                for cb in range(FRAG):
                    f = ftab_v[m * FRAG + cb, sl]
                    for v in range(VOCAB):
                        row = (m * VOCAB + v) * FRAG + cb
                        comb_v[row, sl] = comb_v[row, sl] + f
            return carry

        lax.fori_loop(0, 128 // LANES, build, 0)

        fp_vec = fp_v[pl.ds(0, LANES)]
        phase = fp_vec[0]
        for j in range(1, n_batch):
            phase = jnp.where(b == j, fp_vec[j], phase)

        def out_slice(c):
            return out_hbm.at[
                b, pl.ds(pl.multiple_of(l_base + c * CHUNK, CHUNK), CHUNK)]

        def scatter_wait(c, buf):
            pltpu.make_async_copy(rows_v.at[buf], out_slice(c), ssem).wait()

        def chunk_loop(c, carry):
            buf = c % 2

            @pl.when(c >= 2)
            def _():
                scatter_wait(c - 2, buf)

            for i16 in range(CHUNK // LANES):
                off = c * CHUNK + i16 * LANES
                tok = ids_v[pl.ds(off, LANES)]
                pos = l_base + off + lax.iota(jnp.int32, LANES)
                vrow = (((phase + pos) % NFRAME) * VOCAB + tok) * FRAG
                for j in range(LANES):
                    src = vrow[j]
                    r = i16 * LANES + j
                    for cb in range(FRAG):
                        for g in range(128 // LANES):
                            rows_v[buf, r, pl.ds(cb * 128 + g * LANES,
                                                 LANES)] = (
                                comb_v[src + cb, pl.ds(g * LANES, LANES)])
            pltpu.async_copy(rows_v.at[buf], out_slice(c), ssem)
            return carry

        lax.fori_loop(0, n_chunks, chunk_loop, 0)
        scatter_wait(n_chunks - 2, n_chunks % 2)
        scatter_wait(n_chunks - 1, 1 - n_chunks % 2)

    return k(ids, fp_pad, word2, frame2)


def kernel(ids, frame_phase, word_emb, frame_emb):
    n_batch, seq = ids.shape
    ids32 = ids.astype(jnp.int32)
    fp_pad = jnp.zeros((2 * LANES,), jnp.int32).at[:n_batch].set(
        frame_phase.astype(jnp.int32))
    word2 = word_emb.reshape(VOCAB * FRAG, 128)
    frame2 = frame_emb.reshape(NFRAME * FRAG, 128)
    return _run(ids32, fp_pad, word2, frame2, n_batch, seq)


# speedup vs baseline: 4.3675x; 3.8916x over previous
import functools

import jax
import jax.numpy as jnp
from jax import lax
from jax.experimental import pallas as pl
from jax.experimental.pallas import tpu as pltpu
from jax.experimental.pallas import tpu_sc as plsc

VOCAB = 16
NFRAME = 3
D = 1024
FRAG = D // 128
NC = 2
NS = 16
NW = NC * NS
LANES = 16
PIECES = D // LANES
CHUNK = 32


@functools.partial(jax.jit, static_argnames=("n_batch", "seq"))
def _run(ids, fp_pad, word1, frame1, n_batch, seq):
    n_rows = n_batch * seq
    rows_per_w = n_rows // NW
    n_chunks = rows_per_w // CHUNK
    workers_per_batch = NW // n_batch
    mesh = plsc.VectorSubcoreMesh(
        core_axis_name="c", subcore_axis_name="s",
        num_cores=NC, num_subcores=NS)

    @functools.partial(
        pl.kernel,
        out_type=jax.ShapeDtypeStruct((n_batch, seq, D), jnp.float32),
        mesh=mesh,
        scratch_types=[
            pltpu.VMEM((NFRAME * VOCAB * D,), jnp.float32),
            pltpu.VMEM((NFRAME * D,), jnp.float32),
            pltpu.VMEM((2, CHUNK, D), jnp.float32),
            pltpu.VMEM((2 * LANES,), jnp.int32),
            pltpu.VMEM((rows_per_w,), jnp.int32),
            pltpu.SemaphoreType.DMA,
        ],
    )
    def k(ids_hbm, fp_hbm, word1_hbm, frame1_hbm, out_hbm,
          comb_v, ftab_v, rows_v, fp_v, ids_v, ssem):
        cid = lax.axis_index("c")
        sid = lax.axis_index("s")
        wid = cid * NS + sid
        b = wid // workers_per_batch
        l_base = pl.multiple_of(
            (wid % workers_per_batch) * rows_per_w, rows_per_w)

        for m in range(NFRAME):
            pltpu.sync_copy(
                word1_hbm, comb_v.at[pl.ds(m * VOCAB * D, VOCAB * D)])
        pltpu.sync_copy(frame1_hbm, ftab_v)
        pltpu.sync_copy(ids_hbm.at[b, pl.ds(l_base, rows_per_w)], ids_v)
        pltpu.sync_copy(fp_hbm, fp_v)

        def build(g, carry):
            for m in range(NFRAME):
                for cb in range(FRAG):
                    sl = pl.ds(m * D + cb * 128 + g * LANES, LANES)
                    f = ftab_v[sl]
                    for v in range(VOCAB):
                        dst = pl.ds(
                            (m * VOCAB + v) * D + cb * 128 + g * LANES, LANES)
                        comb_v[dst] = comb_v[dst] + f
            return carry

        lax.fori_loop(0, 128 // LANES, build, 0)

        fp_vec = fp_v[pl.ds(0, LANES)]
        phase = fp_vec[0]
        for j in range(1, n_batch):
            phase = jnp.where(b == j, fp_vec[j], phase)

        def out_slice(c):
            return out_hbm.at[
                b, pl.ds(pl.multiple_of(l_base + c * CHUNK, CHUNK), CHUNK)]

        def scatter_wait(c, buf):
            pltpu.make_async_copy(rows_v.at[buf], out_slice(c), ssem).wait()

        def chunk_loop(c, carry):
            buf = c % 2

            @pl.when(c >= 2)
            def _():
                scatter_wait(c - 2, buf)

            for i16 in range(CHUNK // LANES):
                off = c * CHUNK + i16 * LANES
                tok = ids_v[pl.ds(off, LANES)]
                pos = l_base + off + lax.iota(jnp.int32, LANES)
                cbase = (((phase + pos) % NFRAME) * VOCAB + tok) * D
                for j in range(LANES):
                    src = cbase[j]
                    r = i16 * LANES + j

                    @plsc.parallel_loop(0, PIECES, 1, unroll=8)
                    def piece(p):
                        rows_v[buf, r, pl.ds(p * LANES, LANES)] = (
                            comb_v[pl.ds(src + p * LANES, LANES)])
            pltpu.async_copy(rows_v.at[buf], out_slice(c), ssem)
            return carry

        lax.fori_loop(0, n_chunks, chunk_loop, 0)
        scatter_wait(n_chunks - 2, n_chunks % 2)
        scatter_wait(n_chunks - 1, 1 - n_chunks % 2)

    return k(ids, fp_pad, word1, frame1)


def kernel(ids, frame_phase, word_emb, frame_emb):
    n_batch, seq = ids.shape
    ids32 = ids.astype(jnp.int32)
    fp_pad = jnp.zeros((2 * LANES,), jnp.int32).at[:n_batch].set(
        frame_phase.astype(jnp.int32))
    word1 = word_emb.reshape(VOCAB * D)
    frame1 = frame_emb.reshape(NFRAME * D)
    return _run(ids32, fp_pad, word1, frame1, n_batch, seq)
